# baseline (speedup 1.0000x reference)
import functools

import jax
import jax.numpy as jnp
from jax import lax
from jax.experimental import pallas as pl
from jax.experimental.pallas import tpu as pltpu
from jax.experimental.pallas import tpu_sc as plsc

N = 10000
NP = 10240
E = 160000
EPAD = 163840
IN = 128
H = 1000
HP = 1024
R = 4
P = 1024

NCORES = 2
NSUB = 16
EPT = EPAD // NSUB
KB = 32

f32 = jnp.float32
i32 = jnp.int32

_SC_PARAMS = pltpu.CompilerParams(use_tc_tiling_on_sc=True,
                                  needs_layout_passes=False)


def _make_msg(width: int):
    EPW = EPAD // (NCORES * NSUB)
    mesh = plsc.VectorSubcoreMesh(core_axis_name="c", subcore_axis_name="s",
                                  num_cores=NCORES, num_subcores=NSUB)

    @functools.partial(
        pl.kernel, mesh=mesh, compiler_params=_SC_PARAMS,
        out_type=jax.ShapeDtypeStruct((EPAD, width), f32),
        scratch_types=[
            pltpu.VMEM((KB,), i32),
            pltpu.VMEM((KB, width), f32),
            pltpu.SemaphoreType.DMA,
        ],
    )
    def msg(table_h, gidx_h, msg_h, gib, rows, sem):
        c = lax.axis_index("c")
        s = lax.axis_index("s")
        w = s * NCORES + c
        ebase = w * EPW

        def batch_body(b, carry):
            off = ebase + b * KB
            pltpu.sync_copy(gidx_h.at[pl.ds(off, KB)], gib)
            pltpu.async_copy(table_h.at[gib], rows, sem).wait()
            pltpu.sync_copy(rows, msg_h.at[pl.ds(off, KB)])
            return carry

        lax.fori_loop(0, EPW // KB, batch_body, jnp.int32(0))

    return msg


@functools.lru_cache(maxsize=None)
def _msg0():
    return _make_msg(IN)


@functools.lru_cache(maxsize=None)
def _msgB():
    return _make_msg(HP)


def _scat_body(nrows, si, m, o):
    i = pl.program_id(0)

    @pl.when(i == 0)
    def _():
        o[...] = jnp.zeros_like(o)

    eb = m.shape[0]

    def body(e, carry):
        d = si[0, e]
        o[pl.ds(d, 1), :] += m[pl.ds(e, 1), :]
        return carry

    lax.fori_loop(0, eb, body, jnp.int32(0))


def _scatter_tc(msg, si, nrows, width):
    EB = 1024
    body = functools.partial(_scat_body, nrows)
    return pl.pallas_call(
        body,
        grid=(EPAD // EB,),
        in_specs=[
            pl.BlockSpec((1, EB), lambda i: (0, i),
                         memory_space=pltpu.SMEM),
            pl.BlockSpec((EB, width), lambda i: (i, 0)),
        ],
        out_specs=pl.BlockSpec((nrows, width), lambda i: (0, 0)),
        out_shape=jax.ShapeDtypeStruct((nrows, width), f32),
    )(si.reshape(1, EPAD), msg)


def _add2_body(a, b, o):
    o[...] = a[...] + b[...]


def _add2(a, b):
    BN = 512
    n, w = a.shape
    return pl.pallas_call(
        _add2_body,
        grid=(n // BN,),
        in_specs=[pl.BlockSpec((BN, w), lambda i: (i, 0)),
                  pl.BlockSpec((BN, w), lambda i: (i, 0))],
        out_specs=pl.BlockSpec((BN, w), lambda i: (i, 0)),
        out_shape=jax.ShapeDtypeStruct((n, w), f32),
    )(a, b)


def _make_pair_gather():
    mesh = plsc.VectorSubcoreMesh(core_axis_name="c", subcore_axis_name="s",
                                  num_cores=NCORES, num_subcores=NSUB)
    rows_per = P // (NCORES * NSUB)

    @functools.partial(
        pl.kernel, mesh=mesh, compiler_params=_SC_PARAMS,
        out_type=(jax.ShapeDtypeStruct((P, HP), f32),
                  jax.ShapeDtypeStruct((P, HP), f32)),
        scratch_types=[
            pltpu.VMEM((rows_per,), i32),
            pltpu.VMEM((rows_per, HP), f32),
            pltpu.SemaphoreType.DMA,
        ],
    )
    def pair_gather(h_hbm, i0_h, i1_h, d_out, p_out, ibuf, rows, sem):
        c = lax.axis_index("c")
        s = lax.axis_index("s")
        w = s * NCORES + c
        base = w * rows_per
        pltpu.sync_copy(i0_h.at[pl.ds(base, rows_per)], ibuf)
        pltpu.async_copy(h_hbm.at[ibuf], rows, sem).wait()
        pltpu.sync_copy(rows, d_out.at[pl.ds(base, rows_per)])
        pltpu.sync_copy(i1_h.at[pl.ds(base, rows_per)], ibuf)
        pltpu.async_copy(h_hbm.at[ibuf], rows, sem).wait()
        pltpu.sync_copy(rows, p_out.at[pl.ds(base, rows_per)])

    return pair_gather


_pair_gather = functools.lru_cache(maxsize=None)(_make_pair_gather)


def _combine0_body(a2d, x, wstk, ws, b, o):
    o[...] = (jnp.dot(a2d[...], wstk[...], preferred_element_type=f32)
              + jnp.dot(x[...], ws[...], preferred_element_type=f32)
              + b[...])


def _combine0(A2d, featP, Wstk, Ws0, b0):
    BN = 256
    return pl.pallas_call(
        _combine0_body,
        grid=(NP // BN,),
        in_specs=[
            pl.BlockSpec((BN, R * IN), lambda i: (i, 0)),
            pl.BlockSpec((BN, IN), lambda i: (i, 0)),
            pl.BlockSpec((R * IN, HP), lambda i: (0, 0)),
            pl.BlockSpec((IN, HP), lambda i: (0, 0)),
            pl.BlockSpec((1, HP), lambda i: (0, 0)),
        ],
        out_specs=pl.BlockSpec((BN, HP), lambda i: (i, 0)),
        out_shape=jax.ShapeDtypeStruct((NP, HP), f32),
    )(A2d, featP, Wstk, Ws0, b0)


def _xwself_body(x, wr, ws, b, xw, sw):
    xv = x[...]
    for r in range(R):
        xw[r] = jnp.dot(xv, wr[r], preferred_element_type=f32)
    sw[...] = jnp.dot(xv, ws[...], preferred_element_type=f32) + b[...]


def _xwself(X, Wr, Ws, b):
    BN = 128
    di = X.shape[1]
    return pl.pallas_call(
        _xwself_body,
        grid=(NP // BN,),
        in_specs=[
            pl.BlockSpec((BN, di), lambda i: (i, 0)),
            pl.BlockSpec((R, di, HP), lambda i: (0, 0, 0)),
            pl.BlockSpec((di, HP), lambda i: (0, 0)),
            pl.BlockSpec((1, HP), lambda i: (0, 0)),
        ],
        out_specs=[
            pl.BlockSpec((R, BN, HP), lambda i: (0, i, 0)),
            pl.BlockSpec((BN, HP), lambda i: (i, 0)),
        ],
        out_shape=[jax.ShapeDtypeStruct((R, NP, HP), f32),
                   jax.ShapeDtypeStruct((NP, HP), f32)],
    )(X, Wr, Ws, b)


def _bn_cols(a, g, bt):
    m = jnp.mean(a, axis=0, keepdims=True)
    v = jnp.mean((a - m) ** 2, axis=0, keepdims=True)
    return g * (a - m) / jnp.sqrt(v + 1e-5) + bt


def _mlp0_body(d, p, wa, wb, b, g, bt, o):
    a = jnp.dot(d[...], wa[...], preferred_element_type=f32)
    a = a + jnp.dot(p[...], wb[...], preferred_element_type=f32)
    a = jnp.maximum(a + b[...], 0.0)
    o[...] = _bn_cols(a, g[...], bt[...])


def _mlp0(D, Pr, w0a, w0b, lb0, g0, bt0):
    BD = 512
    DO = 2048
    return pl.pallas_call(
        _mlp0_body,
        grid=(DO // BD,),
        in_specs=[
            pl.BlockSpec((P, HP), lambda i: (0, 0)),
            pl.BlockSpec((P, HP), lambda i: (0, 0)),
            pl.BlockSpec((HP, BD), lambda i: (0, i)),
            pl.BlockSpec((HP, BD), lambda i: (0, i)),
            pl.BlockSpec((1, BD), lambda i: (0, i)),
            pl.BlockSpec((1, BD), lambda i: (0, i)),
            pl.BlockSpec((1, BD), lambda i: (0, i)),
        ],
        out_specs=pl.BlockSpec((P, BD), lambda i: (0, i)),
        out_shape=jax.ShapeDtypeStruct((P, DO), f32),
    )(D, Pr, w0a, w0b, lb0.reshape(1, -1), g0.reshape(1, -1),
      bt0.reshape(1, -1))


def _mlp1_body(x, w, b, g, bt, o):
    a = jnp.maximum(jnp.dot(x[...], w[...], preferred_element_type=f32)
                    + b[...], 0.0)
    o[...] = _bn_cols(a, g[...], bt[...])


def _mlp1(X, lw1, lb1, g1, bt1):
    BD = 512
    DI, DO = 2048, 1024
    return pl.pallas_call(
        _mlp1_body,
        grid=(DO // BD,),
        in_specs=[
            pl.BlockSpec((P, DI), lambda i: (0, 0)),
            pl.BlockSpec((DI, BD), lambda i: (0, i)),
            pl.BlockSpec((1, BD), lambda i: (0, i)),
            pl.BlockSpec((1, BD), lambda i: (0, i)),
            pl.BlockSpec((1, BD), lambda i: (0, i)),
        ],
        out_specs=pl.BlockSpec((P, BD), lambda i: (0, i)),
        out_shape=jax.ShapeDtypeStruct((P, DO), f32),
    )(X, lw1, lb1.reshape(1, -1), g1.reshape(1, -1), bt1.reshape(1, -1))


def _mlp234_body(x, w2, b2, g2v, bt2v, w3, b3, g3v, bt3v, w4, b4, o):
    a = jnp.maximum(jnp.dot(x[...], w2[...], preferred_element_type=f32)
                    + b2[...], 0.0)
    y2 = _bn_cols(a, g2v[...], bt2v[...])
    a = jnp.maximum(jnp.dot(y2, w3[...], preferred_element_type=f32)
                    + b3[...], 0.0)
    y3 = _bn_cols(a, g3v[...], bt3v[...])
    o[...] = jnp.dot(y3, w4[...], preferred_element_type=f32) + b4[...]


def _mlp234(X, lw2, lb2, g2, bt2, lw3, lb3, g3, bt3, lw4, lb4):
    return pl.pallas_call(
        _mlp234_body,
        out_shape=jax.ShapeDtypeStruct((P, 1), f32),
    )(X, lw2, lb2.reshape(1, -1), g2.reshape(1, -1), bt2.reshape(1, -1),
      lw3, lb3.reshape(1, -1), g3.reshape(1, -1), bt3.reshape(1, -1),
      lw4, lb4.reshape(1, -1))


def _padw(w, rows=None, cols=None):
    pr = (0, rows - w.shape[-2]) if rows else (0, 0)
    pc = (0, cols - w.shape[-1]) if cols else (0, 0)
    pad = [(0, 0)] * (w.ndim - 2) + [pr, pc]
    return jnp.pad(w, pad)


def kernel(feat, edge_index, etype, index,
           Wr0, Ws0, b0, Wr1, Ws1, b1, Wr2, Ws2, b2,
           lw0, lb0, lw1, lb1, lw2, lb2, lw3, lb3, lw4, lb4,
           g0, bt0, g1, bt1, g2, bt2, g3, bt3):
    srcP = jnp.pad(edge_index[0], (0, EPAD - E))
    dstP = jnp.pad(edge_index[1], (0, EPAD - E),
                   constant_values=jnp.int32(2 ** 30))
    etP = jnp.pad(etype, (0, EPAD - E))
    edges = jnp.stack([dstP, srcP, etP])
    featP = jnp.zeros((NP, IN), f32).at[:N].set(feat)

    dstC = jnp.minimum(dstP, NP)
    giB = etP * NP + srcP
    siB = dstC

    h = featP
    for li, (Wr, Ws, b) in enumerate(((Wr0, Ws0, b0), (Wr1, Ws1, b1),
                                      (Wr2, Ws2, b2))):
        di = Wr.shape[1]
        rows = HP if li > 0 else None
        xw, selfw = _xwself(h, _padw(Wr, rows=rows, cols=HP),
                            _padw(Ws, rows=rows, cols=HP),
                            _padw(b.reshape(1, H), cols=HP))
        mB = _msgB()(xw.reshape(R * NP, HP), giB)
        agg = _scatter_tc(mB, siB, NP + 8, HP)
        h = _add2(agg[:NP], selfw)

    idxT = index.T
    D, Pr = _pair_gather()(h, idxT[0], idxT[1])
    z = _mlp0(D, Pr, _padw(lw0[:H], rows=HP), _padw(lw0[H:], rows=HP),
              lb0, g0, bt0)
    z = _mlp1(z, lw1, lb1, g1, bt1)
    out = _mlp234(z, lw2, lb2, g2, bt2, lw3, lb3, g3, bt3, lw4, lb4)
    return out, h[:N, :H]


# speedup vs baseline: 1.0262x; 1.0262x over previous
import functools

import jax
import jax.numpy as jnp
from jax import lax
from jax.experimental import pallas as pl
from jax.experimental.pallas import tpu as pltpu
from jax.experimental.pallas import tpu_sc as plsc

N = 10000
NP = 10240
E = 160000
EPAD = 163840
IN = 128
H = 1000
HP = 1024
R = 4
P = 1024

NCORES = 2
NSUB = 16
EPT = EPAD // NSUB
KB = 32

f32 = jnp.float32
i32 = jnp.int32

_SC_PARAMS = pltpu.CompilerParams(use_tc_tiling_on_sc=True,
                                  needs_layout_passes=False)


def _make_msg(width: int):
    EPW = EPAD // (NCORES * NSUB)
    mesh = plsc.VectorSubcoreMesh(core_axis_name="c", subcore_axis_name="s",
                                  num_cores=NCORES, num_subcores=NSUB)

    @functools.partial(
        pl.kernel, mesh=mesh, compiler_params=_SC_PARAMS,
        out_type=jax.ShapeDtypeStruct((EPAD, width), f32),
        scratch_types=[
            pltpu.VMEM((2, KB), i32),
            pltpu.VMEM((KB, width), f32),
            pltpu.VMEM((KB, width), f32),
            pltpu.SemaphoreType.DMA,
            pltpu.SemaphoreType.DMA,
            pltpu.SemaphoreType.DMA,
            pltpu.SemaphoreType.DMA,
        ],
    )
    def msg(table_h, gidx_h, msg_h, gib, rows0, rows1, g0, g1, w0, w1):
        c = lax.axis_index("c")
        s = lax.axis_index("s")
        w = s * NCORES + c
        ebase = w * EPW

        def batch_body(b2, carry):
            off = ebase + b2 * 2 * KB
            pltpu.sync_copy(gidx_h.at[pl.ds(off, KB)], gib.at[0])
            pltpu.sync_copy(gidx_h.at[pl.ds(off + KB, KB)], gib.at[1])
            d0 = pltpu.async_copy(table_h.at[gib.at[0]], rows0, g0)
            d1 = pltpu.async_copy(table_h.at[gib.at[1]], rows1, g1)
            d0.wait()
            wb0 = pltpu.async_copy(rows0, msg_h.at[pl.ds(off, KB)], w0)
            d1.wait()
            wb1 = pltpu.async_copy(rows1, msg_h.at[pl.ds(off + KB, KB)], w1)
            wb0.wait()
            wb1.wait()
            return carry

        lax.fori_loop(0, EPW // (2 * KB), batch_body, jnp.int32(0))

    return msg


@functools.lru_cache(maxsize=None)
def _msg0():
    return _make_msg(IN)


@functools.lru_cache(maxsize=None)
def _msgB():
    return _make_msg(HP)


def _scat_body(nrows, si, m, o):
    i = pl.program_id(0)

    @pl.when(i == 0)
    def _():
        o[...] = jnp.zeros_like(o)

    eb = m.shape[0]

    def body(e, carry):
        d = si[0, e]
        o[pl.ds(d, 1), :] += m[pl.ds(e, 1), :]
        return carry

    lax.fori_loop(0, eb, body, jnp.int32(0))


def _scatter_tc(msg, si, nrows, width):
    EB = 1024
    body = functools.partial(_scat_body, nrows)
    return pl.pallas_call(
        body,
        grid=(EPAD // EB,),
        in_specs=[
            pl.BlockSpec((1, EB), lambda i: (0, i),
                         memory_space=pltpu.SMEM),
            pl.BlockSpec((EB, width), lambda i: (i, 0)),
        ],
        out_specs=pl.BlockSpec((nrows, width), lambda i: (0, 0)),
        out_shape=jax.ShapeDtypeStruct((nrows, width), f32),
    )(si.reshape(1, EPAD), msg)


def _add2_body(a, b, o):
    o[...] = a[...] + b[...]


def _add2(a, b):
    BN = 512
    n, w = a.shape
    return pl.pallas_call(
        _add2_body,
        grid=(n // BN,),
        in_specs=[pl.BlockSpec((BN, w), lambda i: (i, 0)),
                  pl.BlockSpec((BN, w), lambda i: (i, 0))],
        out_specs=pl.BlockSpec((BN, w), lambda i: (i, 0)),
        out_shape=jax.ShapeDtypeStruct((n, w), f32),
    )(a, b)


def _make_pair_gather():
    mesh = plsc.VectorSubcoreMesh(core_axis_name="c", subcore_axis_name="s",
                                  num_cores=NCORES, num_subcores=NSUB)
    rows_per = P // (NCORES * NSUB)

    @functools.partial(
        pl.kernel, mesh=mesh, compiler_params=_SC_PARAMS,
        out_type=(jax.ShapeDtypeStruct((P, HP), f32),
                  jax.ShapeDtypeStruct((P, HP), f32)),
        scratch_types=[
            pltpu.VMEM((rows_per,), i32),
            pltpu.VMEM((rows_per, HP), f32),
            pltpu.SemaphoreType.DMA,
        ],
    )
    def pair_gather(h_hbm, i0_h, i1_h, d_out, p_out, ibuf, rows, sem):
        c = lax.axis_index("c")
        s = lax.axis_index("s")
        w = s * NCORES + c
        base = w * rows_per
        pltpu.sync_copy(i0_h.at[pl.ds(base, rows_per)], ibuf)
        pltpu.async_copy(h_hbm.at[ibuf], rows, sem).wait()
        pltpu.sync_copy(rows, d_out.at[pl.ds(base, rows_per)])
        pltpu.sync_copy(i1_h.at[pl.ds(base, rows_per)], ibuf)
        pltpu.async_copy(h_hbm.at[ibuf], rows, sem).wait()
        pltpu.sync_copy(rows, p_out.at[pl.ds(base, rows_per)])

    return pair_gather


_pair_gather = functools.lru_cache(maxsize=None)(_make_pair_gather)


def _combine0_body(a2d, x, wstk, ws, b, o):
    o[...] = (jnp.dot(a2d[...], wstk[...], preferred_element_type=f32)
              + jnp.dot(x[...], ws[...], preferred_element_type=f32)
              + b[...])


def _combine0(A2d, featP, Wstk, Ws0, b0):
    BN = 256
    return pl.pallas_call(
        _combine0_body,
        grid=(NP // BN,),
        in_specs=[
            pl.BlockSpec((BN, R * IN), lambda i: (i, 0)),
            pl.BlockSpec((BN, IN), lambda i: (i, 0)),
            pl.BlockSpec((R * IN, HP), lambda i: (0, 0)),
            pl.BlockSpec((IN, HP), lambda i: (0, 0)),
            pl.BlockSpec((1, HP), lambda i: (0, 0)),
        ],
        out_specs=pl.BlockSpec((BN, HP), lambda i: (i, 0)),
        out_shape=jax.ShapeDtypeStruct((NP, HP), f32),
    )(A2d, featP, Wstk, Ws0, b0)


def _xwself_body(x, wr, ws, b, xw, sw):
    xv = x[...]
    for r in range(R):
        xw[r] = jnp.dot(xv, wr[r], preferred_element_type=f32)
    sw[...] = jnp.dot(xv, ws[...], preferred_element_type=f32) + b[...]


def _xwself(X, Wr, Ws, b):
    BN = 128
    di = X.shape[1]
    return pl.pallas_call(
        _xwself_body,
        grid=(NP // BN,),
        in_specs=[
            pl.BlockSpec((BN, di), lambda i: (i, 0)),
            pl.BlockSpec((R, di, HP), lambda i: (0, 0, 0)),
            pl.BlockSpec((di, HP), lambda i: (0, 0)),
            pl.BlockSpec((1, HP), lambda i: (0, 0)),
        ],
        out_specs=[
            pl.BlockSpec((R, BN, HP), lambda i: (0, i, 0)),
            pl.BlockSpec((BN, HP), lambda i: (i, 0)),
        ],
        out_shape=[jax.ShapeDtypeStruct((R, NP, HP), f32),
                   jax.ShapeDtypeStruct((NP, HP), f32)],
    )(X, Wr, Ws, b)


def _bn_cols(a, g, bt):
    m = jnp.mean(a, axis=0, keepdims=True)
    v = jnp.mean((a - m) ** 2, axis=0, keepdims=True)
    return g * (a - m) / jnp.sqrt(v + 1e-5) + bt


def _mlp0_body(d, p, wa, wb, b, g, bt, o):
    a = jnp.dot(d[...], wa[...], preferred_element_type=f32)
    a = a + jnp.dot(p[...], wb[...], preferred_element_type=f32)
    a = jnp.maximum(a + b[...], 0.0)
    o[...] = _bn_cols(a, g[...], bt[...])


def _mlp0(D, Pr, w0a, w0b, lb0, g0, bt0):
    BD = 512
    DO = 2048
    return pl.pallas_call(
        _mlp0_body,
        grid=(DO // BD,),
        in_specs=[
            pl.BlockSpec((P, HP), lambda i: (0, 0)),
            pl.BlockSpec((P, HP), lambda i: (0, 0)),
            pl.BlockSpec((HP, BD), lambda i: (0, i)),
            pl.BlockSpec((HP, BD), lambda i: (0, i)),
            pl.BlockSpec((1, BD), lambda i: (0, i)),
            pl.BlockSpec((1, BD), lambda i: (0, i)),
            pl.BlockSpec((1, BD), lambda i: (0, i)),
        ],
        out_specs=pl.BlockSpec((P, BD), lambda i: (0, i)),
        out_shape=jax.ShapeDtypeStruct((P, DO), f32),
    )(D, Pr, w0a, w0b, lb0.reshape(1, -1), g0.reshape(1, -1),
      bt0.reshape(1, -1))


def _mlp1_body(x, w, b, g, bt, o):
    a = jnp.maximum(jnp.dot(x[...], w[...], preferred_element_type=f32)
                    + b[...], 0.0)
    o[...] = _bn_cols(a, g[...], bt[...])


def _mlp1(X, lw1, lb1, g1, bt1):
    BD = 512
    DI, DO = 2048, 1024
    return pl.pallas_call(
        _mlp1_body,
        grid=(DO // BD,),
        in_specs=[
            pl.BlockSpec((P, DI), lambda i: (0, 0)),
            pl.BlockSpec((DI, BD), lambda i: (0, i)),
            pl.BlockSpec((1, BD), lambda i: (0, i)),
            pl.BlockSpec((1, BD), lambda i: (0, i)),
            pl.BlockSpec((1, BD), lambda i: (0, i)),
        ],
        out_specs=pl.BlockSpec((P, BD), lambda i: (0, i)),
        out_shape=jax.ShapeDtypeStruct((P, DO), f32),
    )(X, lw1, lb1.reshape(1, -1), g1.reshape(1, -1), bt1.reshape(1, -1))


def _mlp234_body(x, w2, b2, g2v, bt2v, w3, b3, g3v, bt3v, w4, b4, o):
    a = jnp.maximum(jnp.dot(x[...], w2[...], preferred_element_type=f32)
                    + b2[...], 0.0)
    y2 = _bn_cols(a, g2v[...], bt2v[...])
    a = jnp.maximum(jnp.dot(y2, w3[...], preferred_element_type=f32)
                    + b3[...], 0.0)
    y3 = _bn_cols(a, g3v[...], bt3v[...])
    o[...] = jnp.dot(y3, w4[...], preferred_element_type=f32) + b4[...]


def _mlp234(X, lw2, lb2, g2, bt2, lw3, lb3, g3, bt3, lw4, lb4):
    return pl.pallas_call(
        _mlp234_body,
        out_shape=jax.ShapeDtypeStruct((P, 1), f32),
    )(X, lw2, lb2.reshape(1, -1), g2.reshape(1, -1), bt2.reshape(1, -1),
      lw3, lb3.reshape(1, -1), g3.reshape(1, -1), bt3.reshape(1, -1),
      lw4, lb4.reshape(1, -1))


def _padw(w, rows=None, cols=None):
    pr = (0, rows - w.shape[-2]) if rows else (0, 0)
    pc = (0, cols - w.shape[-1]) if cols else (0, 0)
    pad = [(0, 0)] * (w.ndim - 2) + [pr, pc]
    return jnp.pad(w, pad)


def kernel(feat, edge_index, etype, index,
           Wr0, Ws0, b0, Wr1, Ws1, b1, Wr2, Ws2, b2,
           lw0, lb0, lw1, lb1, lw2, lb2, lw3, lb3, lw4, lb4,
           g0, bt0, g1, bt1, g2, bt2, g3, bt3):
    srcP = jnp.pad(edge_index[0], (0, EPAD - E))
    dstP = jnp.pad(edge_index[1], (0, EPAD - E),
                   constant_values=jnp.int32(2 ** 30))
    etP = jnp.pad(etype, (0, EPAD - E))
    edges = jnp.stack([dstP, srcP, etP])
    featP = jnp.zeros((NP, IN), f32).at[:N].set(feat)

    dstC = jnp.minimum(dstP, NP)
    giB = etP * NP + srcP
    siB = dstC

    h = featP
    for li, (Wr, Ws, b) in enumerate(((Wr0, Ws0, b0), (Wr1, Ws1, b1),
                                      (Wr2, Ws2, b2))):
        di = Wr.shape[1]
        rows = HP if li > 0 else None
        xw, selfw = _xwself(h, _padw(Wr, rows=rows, cols=HP),
                            _padw(Ws, rows=rows, cols=HP),
                            _padw(b.reshape(1, H), cols=HP))
        mB = _msgB()(xw.reshape(R * NP, HP), giB)
        agg = _scatter_tc(mB, siB, NP + 8, HP)
        h = _add2(agg[:NP], selfw)

    idxT = index.T
    D, Pr = _pair_gather()(h, idxT[0], idxT[1])
    z = _mlp0(D, Pr, _padw(lw0[:H], rows=HP), _padw(lw0[H:], rows=HP),
              lb0, g0, bt0)
    z = _mlp1(z, lw1, lb1, g1, bt1)
    out = _mlp234(z, lw2, lb2, g2, bt2, lw3, lb3, g3, bt3, lw4, lb4)
    return out, h[:N, :H]
